# baseline (speedup 1.0000x reference)
import functools
import math

import jax
import jax.numpy as jnp
from jax import lax
from jax.experimental import pallas as pl
from jax.experimental.pallas import tpu as pltpu

_BETA = 0.6
_ALPHA = 0.25
_EPS = 1e-5
_E = math.e
_LOG2E = math.log2(math.e)
_VMEM_LIMIT = 48 * 1024 * 1024


def _pick_rb(n):
    for c in (768, 512, 384, 256, 128):
        if n % c == 0:
            return c
    return n


def _prelu(x):
    return jnp.where(x >= 0.0, x, _ALPHA * x)


def _bf16(x):
    return x.astype(jnp.bfloat16)


def _f32(x):
    return x.astype(jnp.float32)


def _merit_kernel(adj_ref, feat_ref, wpk_ref, o_ref,
                  xw_ref, rep_ref, pred_ref, tproj_ref, predsc_ref,
                  *, n, f, d, rbe, nbe, rbl, nbl):
    s = pl.program_id(0)
    v = jnp.minimum(s // nbe, 1)
    r = s % nbe
    m0 = 2 * f
    v0 = 2 * f + 6 * d

    @pl.when((s < 2 * nbe) & (r == 0))
    def _():
        ft = feat_ref[v]
        xw_ref[:, :d] = jnp.dot(ft, wpk_ref[0:f, :],
                                preferred_element_type=jnp.float32)
        xw_ref[:, d:] = jnp.dot(ft, wpk_ref[f:2 * f, :],
                                preferred_element_type=jnp.float32)

    @pl.when(s < 2 * nbe)
    def _():
        bias = jnp.concatenate([wpk_ref[v0:v0 + 1, :],
                                wpk_ref[v0 + 1:v0 + 2, :]], axis=1)
        gb = jnp.dot(adj_ref[0], xw_ref[...],
                     preferred_element_type=jnp.float32)
        rep_ref[pl.ds(r * rbe, rbe), :] = _prelu(gb + bias)

    @pl.when((s < 2 * nbe) & (r == nbe - 1))
    def _():
        def mlp(x, wrow, vrow):
            y = jnp.dot(x, wpk_ref[wrow:wrow + d, :],
                        preferred_element_type=jnp.float32)
            y = y + wpk_ref[vrow:vrow + 1, :]
            s1 = jnp.sum(y, axis=0, keepdims=True)
            s2 = jnp.sum(y * y, axis=0, keepdims=True)
            mu = s1 * (1.0 / n)
            var = s2 * (1.0 / n) - mu * mu
            k1 = lax.rsqrt(var + _EPS) * wpk_ref[vrow + 1:vrow + 2, :]
            k2 = wpk_ref[vrow + 2:vrow + 3, :] - mu * k1
            z = _prelu(y * k1 + k2)
            return (jnp.dot(z, wpk_ref[wrow + d:wrow + 2 * d, :],
                            preferred_element_type=jnp.float32)
                    + wpk_ref[vrow + 3:vrow + 4, :])

        def unit(x):
            ss = jnp.sum(x * x, axis=-1, keepdims=True)
            return x * lax.rsqrt(jnp.maximum(ss, 1e-24))

        o_proj = mlp(rep_ref[:, :d], m0, v0 + 2)
        o_pred = mlp(o_proj, m0 + 2 * d, v0 + 6)
        t_proj = mlp(rep_ref[:, d:], m0 + 4 * d, v0 + 10)
        hu = unit(o_pred)
        vrows = pl.ds(v * n, n)
        pred_ref[vrows, :] = _bf16(hu)
        predsc_ref[vrows, :] = _bf16(hu * _LOG2E)
        tproj_ref[vrows, :] = _bf16(unit(t_proj))

    @pl.when(s == 2 * nbe)
    def _():
        o_ref[...] = jnp.zeros_like(o_ref)

    @pl.when(s >= 2 * nbe)
    def _():
        b0 = (s - 2 * nbe) * rbl
        h1b = pred_ref[pl.ds(b0, rbl), :]
        h2b = pred_ref[pl.ds(n + b0, rbl), :]
        h1bs = predsc_ref[pl.ds(b0, rbl), :]
        h2bs = predsc_ref[pl.ds(n + b0, rbl), :]
        z1b = tproj_ref[pl.ds(b0, rbl), :]
        z2b = tproj_ref[pl.ds(n + b0, rbl), :]

        def rsum(m):
            return jnp.sum(m, axis=-1, keepdims=True)

        def csum(x):
            return jnp.sum(rsum(x), axis=0, keepdims=True)

        def expsum(a, c_ref, r0, nrows):
            acc = None
            for k in range(0, nrows, 256):
                ck = c_ref[r0 + k:r0 + k + 256, :]
                sm = lax.dot_general(a, ck, (((1,), (1,)), ((), ())),
                                     preferred_element_type=jnp.float32)
                p = rsum(jnp.exp2(sm))
                acc = p if acc is None else acc + p
            return acc

        den1 = expsum(h1bs, pred_ref, 0, 2 * n) - _E
        den2 = expsum(h2bs, pred_ref, 0, 2 * n) - _E
        net = csum(jnp.log(den1)) + csum(jnp.log(den2))
        view = (csum(jnp.log(expsum(h1bs, tproj_ref, n, n))) +
                csum(jnp.log(expsum(h2bs, tproj_ref, 0, n))))

        h1f = _f32(h1b)
        h2f = _f32(h2b)
        dots = (2.0 * _BETA * csum(h1f * _f32(h2b))
                + (1.0 - _BETA) * (csum(h1f * _f32(z2b))
                                   + csum(h2f * _f32(z1b))))
        part = _BETA * net + (1.0 - _BETA) * view - dots
        o_ref[...] += part * (0.5 / n)


def _merit_forward(adj, feat, wpk):
    _, n, f = feat.shape
    d = wpk.shape[-1]
    rbe = 768 if n % 768 == 0 else _pick_rb(n)
    nbe = n // rbe
    rbl = _pick_rb(n)
    nbl = n // rbl
    body = functools.partial(_merit_kernel, n=n, f=f, d=d,
                             rbe=rbe, nbe=nbe, rbl=rbl, nbl=nbl)
    in_specs = [
        pl.BlockSpec((1, rbe, n),
                     lambda s: (jnp.minimum(s // nbe, 1),
                                jnp.where(s < 2 * nbe, s % nbe, nbe - 1), 0)),
        pl.BlockSpec((2, n, f), lambda s: (0, 0, 0)),
        pl.BlockSpec(wpk.shape, lambda s: (0, 0)),
    ]
    out = pl.pallas_call(
        body,
        grid=(2 * nbe + nbl,),
        in_specs=in_specs,
        out_specs=pl.BlockSpec((1, 1), lambda s: (0, 0)),
        out_shape=jax.ShapeDtypeStruct((1, 1), jnp.float32),
        scratch_shapes=[pltpu.VMEM((n, 2 * d), jnp.float32),
                        pltpu.VMEM((n, 2 * d), jnp.float32),
                        pltpu.VMEM((2 * n, d), jnp.bfloat16),
                        pltpu.VMEM((2 * n, d), jnp.bfloat16),
                        pltpu.VMEM((2 * n, d), jnp.bfloat16)],
        compiler_params=pltpu.CompilerParams(
            dimension_semantics=("arbitrary",),
            allow_input_fusion=[False, False, True],
            vmem_limit_bytes=_VMEM_LIMIT),
    )(adj, feat, wpk)
    return out[0, 0]


def kernel(adj, feat,
           online_gcn_w, online_gcn_b,
           online_proj_w1, online_proj_b1, online_proj_gamma,
           online_proj_beta, online_proj_w2, online_proj_b2,
           target_gcn_w, target_gcn_b,
           target_proj_w1, target_proj_b1, target_proj_gamma,
           target_proj_beta, target_proj_w2, target_proj_b2,
           pred_w1, pred_b1, pred_gamma, pred_beta, pred_w2, pred_b2):
    wpk = jnp.concatenate([
        online_gcn_w, target_gcn_w,
        online_proj_w1, online_proj_w2,
        pred_w1, pred_w2,
        target_proj_w1, target_proj_w2,
        online_gcn_b, target_gcn_b,
        online_proj_b1, online_proj_gamma, online_proj_beta, online_proj_b2,
        pred_b1, pred_gamma, pred_beta, pred_b2,
        target_proj_b1, target_proj_gamma, target_proj_beta, target_proj_b2,
    ], axis=0)
    return _merit_forward(adj, feat, wpk)


# speedup vs baseline: 1.0535x; 1.0535x over previous
import functools
import math

import jax
import jax.numpy as jnp
from jax import lax
from jax.experimental import pallas as pl
from jax.experimental.pallas import tpu as pltpu

_BETA = 0.6
_ALPHA = 0.25
_EPS = 1e-5
_E = math.e
_LOG2E = math.log2(math.e)
_VMEM_LIMIT = 48 * 1024 * 1024


def _pick_rb(n):
    for c in (768, 512, 384, 256, 128):
        if n % c == 0:
            return c
    return n


def _prelu(x):
    return jnp.where(x >= 0.0, x, _ALPHA * x)


def _bf16(x):
    return x.astype(jnp.bfloat16)


def _f32(x):
    return x.astype(jnp.float32)


def _merit_kernel(adj_ref, feat_ref, wpk_ref, o_ref,
                  xw_ref, rep_ref, pred_ref, tproj_ref, predsc_ref,
                  *, n, f, d, rbe, nbe, rbl, nbl):
    s = pl.program_id(0)
    v = jnp.minimum(s // nbe, 1)
    r = s % nbe
    m0 = 2 * f
    v0 = 2 * f + 6 * d

    @pl.when((s < 2 * nbe) & (r == 0))
    def _():
        ft = feat_ref[v]
        xw_ref[:, :d] = jnp.dot(ft, wpk_ref[0:f, :],
                                preferred_element_type=jnp.float32)
        xw_ref[:, d:] = jnp.dot(ft, wpk_ref[f:2 * f, :],
                                preferred_element_type=jnp.float32)

    @pl.when(s < 2 * nbe)
    def _():
        bias = jnp.concatenate([wpk_ref[v0:v0 + 1, :],
                                wpk_ref[v0 + 1:v0 + 2, :]], axis=1)
        gb = jnp.dot(adj_ref[0], xw_ref[...],
                     preferred_element_type=jnp.float32)
        rep_ref[pl.ds(r * rbe, rbe), :] = _prelu(gb + bias)

    @pl.when((s < 2 * nbe) & (r == nbe - 1))
    def _():
        def mlp(x, wrow, vrow):
            y = jnp.dot(x, wpk_ref[wrow:wrow + d, :],
                        preferred_element_type=jnp.float32)
            s1 = jnp.sum(y, axis=0, keepdims=True)
            s2 = jnp.sum(y * y, axis=0, keepdims=True)
            mu = s1 * (1.0 / n)
            var = s2 * (1.0 / n) - mu * mu
            k1 = lax.rsqrt(var + _EPS) * wpk_ref[vrow + 1:vrow + 2, :]
            k2 = wpk_ref[vrow + 2:vrow + 3, :] - mu * k1
            z = _prelu(y * k1 + k2)
            return (jnp.dot(z, wpk_ref[wrow + d:wrow + 2 * d, :],
                            preferred_element_type=jnp.float32)
                    + wpk_ref[vrow + 3:vrow + 4, :])

        def unit(x):
            ss = jnp.sum(x * x, axis=-1, keepdims=True)
            return x * lax.rsqrt(jnp.maximum(ss, 1e-24))

        o_proj = mlp(rep_ref[:, :d], m0, v0 + 2)
        o_pred = mlp(o_proj, m0 + 2 * d, v0 + 6)
        t_proj = mlp(rep_ref[:, d:], m0 + 4 * d, v0 + 10)
        hu = unit(o_pred)
        vrows = pl.ds(v * n, n)
        pred_ref[vrows, :] = _bf16(hu)
        predsc_ref[vrows, :] = _bf16(hu * _LOG2E)
        tproj_ref[vrows, :] = _bf16(unit(t_proj))

    @pl.when(s == 2 * nbe)
    def _():
        o_ref[...] = jnp.zeros_like(o_ref)

    @pl.when(s >= 2 * nbe)
    def _():
        b0 = (s - 2 * nbe) * rbl
        h1b = pred_ref[pl.ds(b0, rbl), :]
        h2b = pred_ref[pl.ds(n + b0, rbl), :]
        h1bs = predsc_ref[pl.ds(b0, rbl), :]
        h2bs = predsc_ref[pl.ds(n + b0, rbl), :]
        z1b = tproj_ref[pl.ds(b0, rbl), :]
        z2b = tproj_ref[pl.ds(n + b0, rbl), :]

        def expdot(a, c):
            sm = lax.dot_general(a, c, (((1,), (1,)), ((), ())),
                                 preferred_element_type=jnp.float32)
            return jnp.exp2(sm)

        def rsum(m):
            return jnp.sum(m, axis=-1, keepdims=True)

        def csum(x):
            return jnp.sum(rsum(x), axis=0, keepdims=True)

        hh = pred_ref[...]
        z1 = tproj_ref[0:n, :]
        z2 = tproj_ref[n:2 * n, :]
        den1 = rsum(expdot(h1bs, hh)) - _E
        den2 = rsum(expdot(h2bs, hh)) - _E
        net = csum(jnp.log(den1)) + csum(jnp.log(den2))
        view = (csum(jnp.log(rsum(expdot(h1bs, z2)))) +
                csum(jnp.log(rsum(expdot(h2bs, z1)))))

        h1f = _f32(h1b)
        h2f = _f32(h2b)
        dots = (2.0 * _BETA * csum(h1f * _f32(h2b))
                + (1.0 - _BETA) * (csum(h1f * _f32(z2b))
                                   + csum(h2f * _f32(z1b))))
        part = _BETA * net + (1.0 - _BETA) * view - dots
        o_ref[...] += part * (0.5 / n)


def _merit_forward(adj, feat, wpk):
    _, n, f = feat.shape
    d = wpk.shape[-1]
    rbe = 768 if n % 768 == 0 else _pick_rb(n)
    nbe = n // rbe
    rbl = _pick_rb(n)
    nbl = n // rbl
    body = functools.partial(_merit_kernel, n=n, f=f, d=d,
                             rbe=rbe, nbe=nbe, rbl=rbl, nbl=nbl)
    in_specs = [
        pl.BlockSpec((1, rbe, n),
                     lambda s: (jnp.minimum(s // nbe, 1),
                                jnp.where(s < 2 * nbe, s % nbe, nbe - 1), 0)),
        pl.BlockSpec((2, n, f), lambda s: (0, 0, 0)),
        pl.BlockSpec(wpk.shape, lambda s: (0, 0)),
    ]
    out = pl.pallas_call(
        body,
        grid=(2 * nbe + nbl,),
        in_specs=in_specs,
        out_specs=pl.BlockSpec((1, 1), lambda s: (0, 0)),
        out_shape=jax.ShapeDtypeStruct((1, 1), jnp.float32),
        scratch_shapes=[pltpu.VMEM((n, 2 * d), jnp.float32),
                        pltpu.VMEM((n, 2 * d), jnp.float32),
                        pltpu.VMEM((2 * n, d), jnp.bfloat16),
                        pltpu.VMEM((2 * n, d), jnp.bfloat16),
                        pltpu.VMEM((2 * n, d), jnp.bfloat16)],
        compiler_params=pltpu.CompilerParams(
            dimension_semantics=("arbitrary",),
            allow_input_fusion=[False, False, True],
            vmem_limit_bytes=_VMEM_LIMIT),
    )(adj, feat, wpk)
    return out[0, 0]


def kernel(adj, feat,
           online_gcn_w, online_gcn_b,
           online_proj_w1, online_proj_b1, online_proj_gamma,
           online_proj_beta, online_proj_w2, online_proj_b2,
           target_gcn_w, target_gcn_b,
           target_proj_w1, target_proj_b1, target_proj_gamma,
           target_proj_beta, target_proj_w2, target_proj_b2,
           pred_w1, pred_b1, pred_gamma, pred_beta, pred_w2, pred_b2):
    wpk = jnp.concatenate([
        online_gcn_w, target_gcn_w,
        online_proj_w1, online_proj_w2,
        pred_w1, pred_w2,
        target_proj_w1, target_proj_w2,
        online_gcn_b, target_gcn_b,
        online_proj_b1, online_proj_gamma, online_proj_beta, online_proj_b2,
        pred_b1, pred_gamma, pred_beta, pred_b2,
        target_proj_b1, target_proj_gamma, target_proj_beta, target_proj_b2,
    ], axis=0)
    return _merit_forward(adj, feat, wpk)
